# baseline (speedup 1.0000x reference)
import jax
import jax.numpy as jnp
from jax import lax
from jax.experimental import pallas as pl
from jax.experimental.pallas import tpu as pltpu
from jax.experimental.pallas import tpu_sc as plsc

VOCAB = 1000000
EMBED_DIM = 64
NUM_CLASSES = 2
BATCH = 4096
HIST = 200

NC = 2
NS = 16
NW = NC * NS
ROWS_PER_W = BATCH // NW
CHUNK = 100
CHUNKS_PER_ROW = HIST // CHUNK
IDX_ROWS = ROWS_PER_W * CHUNKS_PER_ROW
D16 = EMBED_DIM // 16


def _sc_body(table_hbm, x2_hbm, wt_hbm, b_hbm, out_hbm,
             idx_v, buf_a, buf_b, wt_v, b_v, out_v, sem_a, sem_b):
    cid = lax.axis_index("c")
    sid = lax.axis_index("s")
    wid = sid * NC + cid

    pltpu.sync_copy(x2_hbm.at[pl.ds(wid * IDX_ROWS, IDX_ROWS)], idx_v)
    pltpu.sync_copy(wt_hbm, wt_v)
    pltpu.sync_copy(b_hbm, b_v)

    lane = lax.broadcasted_iota(jnp.int32, (16,), 0)
    zero = jnp.zeros((16,), jnp.float32)
    b_vec = b_v[...]
    inv_l = jnp.float32(1.0 / HIST)

    def reduce_buf(buf, acc):
        def body(r, a):
            return tuple(a[k] + buf[r, pl.ds(k * 16, 16)] for k in range(D16))
        return lax.fori_loop(0, CHUNK, body, acc)

    def row_body(i, carry):
        del carry
        cp_a = pltpu.async_copy(table_hbm.at[idx_v.at[2 * i]], buf_a, sem_a)
        cp_b = pltpu.async_copy(table_hbm.at[idx_v.at[2 * i + 1]], buf_b, sem_b)
        acc = (zero,) * D16
        cp_a.wait()
        acc = reduce_buf(buf_a, acc)
        cp_b.wait()
        acc = reduce_buf(buf_b, acc)
        out_row = b_vec
        for c in range(NUM_CLASSES):
            s = jnp.float32(0.0)
            for k in range(D16):
                s = s + jnp.sum(acc[k] * wt_v[c, pl.ds(k * 16, 16)])
            s = s * inv_l
            out_row = out_row + jnp.where(lane == c, s, 0.0)
        out_v[i] = out_row
        return 0

    lax.fori_loop(0, ROWS_PER_W, row_body, 0)
    pltpu.sync_copy(out_v, out_hbm.at[pl.ds(wid * ROWS_PER_W, ROWS_PER_W)])


_sc_call = pl.kernel(
    _sc_body,
    out_type=jax.ShapeDtypeStruct((BATCH, 16), jnp.float32),
    mesh=plsc.VectorSubcoreMesh(core_axis_name="c", subcore_axis_name="s"),
    scratch_types=[
        pltpu.VMEM((IDX_ROWS, CHUNK), jnp.int32),
        pltpu.VMEM((CHUNK, EMBED_DIM), jnp.float32),
        pltpu.VMEM((CHUNK, EMBED_DIM), jnp.float32),
        pltpu.VMEM((NUM_CLASSES, EMBED_DIM), jnp.float32),
        pltpu.VMEM((16,), jnp.float32),
        pltpu.VMEM((ROWS_PER_W, 16), jnp.float32),
        pltpu.SemaphoreType.DMA,
        pltpu.SemaphoreType.DMA,
    ],
    compiler_params=pltpu.CompilerParams(
        needs_layout_passes=False, use_tc_tiling_on_sc=False),
)


@jax.jit
def kernel(x, table, W, b):
    x2 = x.astype(jnp.int32).reshape(BATCH * CHUNKS_PER_ROW, CHUNK)
    wt = W.T.astype(jnp.float32)
    b_pad = jnp.pad(b.astype(jnp.float32), (0, 16 - NUM_CLASSES))
    out16 = _sc_call(table, x2, wt, b_pad)
    return out16[:, :NUM_CLASSES]


# speedup vs baseline: 1.1907x; 1.1907x over previous
import jax
import jax.numpy as jnp
from jax import lax
from jax.experimental import pallas as pl
from jax.experimental.pallas import tpu as pltpu
from jax.experimental.pallas import tpu_sc as plsc

VOCAB = 1000000
EMBED_DIM = 64
NUM_CLASSES = 2
BATCH = 4096
HIST = 200

NC = 2
NS = 16
NW = NC * NS
ROWS_PER_W = BATCH // NW
CHUNK = 100
CHUNKS_PER_ROW = HIST // CHUNK
IDX_ROWS = ROWS_PER_W * CHUNKS_PER_ROW
D16 = EMBED_DIM // 16
NBUF = 4


def _sc_body(table_hbm, x2_hbm, wt_hbm, b_hbm, out_hbm,
             idx_v, bufs, wt_v, b_v, out_v, sems):
    cid = lax.axis_index("c")
    sid = lax.axis_index("s")
    wid = sid * NC + cid

    pltpu.sync_copy(x2_hbm.at[pl.ds(wid * IDX_ROWS, IDX_ROWS)], idx_v)
    pltpu.sync_copy(wt_hbm, wt_v)
    pltpu.sync_copy(b_hbm, b_v)

    lane = lax.broadcasted_iota(jnp.int32, (16,), 0)
    zero = jnp.zeros((16,), jnp.float32)
    b_vec = b_v[...]
    wvecs = tuple(wt_v[c, pl.ds(k * 16, 16)]
                  for c in range(NUM_CLASSES) for k in range(D16))
    inv_l = jnp.float32(1.0 / HIST)

    def fire(slot, j):
        return pltpu.async_copy(table_hbm.at[idx_v.at[j]], bufs.at[slot],
                                sems.at[slot])

    def wait(slot):
        pltpu.make_async_copy(table_hbm.at[idx_v.at[0]], bufs.at[slot],
                              sems.at[slot]).wait()

    def reduce_buf(slot, acc):
        buf = bufs.at[slot]

        @plsc.parallel_loop(0, CHUNK, step=1, unroll=4, carry=acc)
        def body(r, a):
            return tuple(a[k] + buf[r, pl.ds(k * 16, 16)] for k in range(D16))

        return body

    def finalize(row, acc):
        out_row = b_vec
        for c in range(NUM_CLASSES):
            s = jnp.float32(0.0)
            for k in range(D16):
                s = s + jnp.sum(acc[k] * wvecs[c * D16 + k])
            out_row = out_row + jnp.where(lane == c, s * inv_l, 0.0)
        out_v[row] = out_row

    for b in range(NBUF):
        fire(b, b)

    @pl.loop(0, IDX_ROWS - NBUF, step=NBUF)
    def _(g):
        row = g >> 1
        for b in range(NBUF):
            wait(b)
            acc = (zero,) * D16 if b % 2 == 0 else acc2
            acc2 = reduce_buf(b, acc)
            fire(b, g + b + NBUF)
            if b % 2 == 1:
                finalize(row + b // 2, acc2)

    for b in range(NBUF):
        j = IDX_ROWS - NBUF + b
        wait(b)
        acc = (zero,) * D16 if b % 2 == 0 else acc2
        acc2 = reduce_buf(b, acc)
        if b % 2 == 1:
            finalize((IDX_ROWS - NBUF + b) // 2, acc2)

    pltpu.sync_copy(out_v, out_hbm.at[pl.ds(wid * ROWS_PER_W, ROWS_PER_W)])


_sc_call = pl.kernel(
    _sc_body,
    out_type=jax.ShapeDtypeStruct((BATCH, 16), jnp.float32),
    mesh=plsc.VectorSubcoreMesh(core_axis_name="c", subcore_axis_name="s"),
    scratch_types=[
        pltpu.VMEM((IDX_ROWS, CHUNK), jnp.int32),
        pltpu.VMEM((NBUF, CHUNK, EMBED_DIM), jnp.float32),
        pltpu.VMEM((NUM_CLASSES, EMBED_DIM), jnp.float32),
        pltpu.VMEM((16,), jnp.float32),
        pltpu.VMEM((ROWS_PER_W, 16), jnp.float32),
        pltpu.SemaphoreType.DMA((NBUF,)),
    ],
    compiler_params=pltpu.CompilerParams(
        needs_layout_passes=False, use_tc_tiling_on_sc=False),
)


@jax.jit
def kernel(x, table, W, b):
    x2 = x.astype(jnp.int32).reshape(BATCH * CHUNKS_PER_ROW, CHUNK)
    wt = W.T.astype(jnp.float32)
    b_pad = jnp.pad(b.astype(jnp.float32), (0, 16 - NUM_CLASSES))
    out16 = _sc_call(table, x2, wt, b_pad)
    return out16[:, :NUM_CLASSES]
